# baseline (speedup 1.0000x reference)
import functools

import jax
import jax.numpy as jnp
from jax import lax
from jax.experimental import pallas as pl
from jax.experimental.pallas import tpu as pltpu
from jax.experimental.pallas import tpu_sc as plsc

EMBED = 64
BATCH = 16384
CHUNK = 128


def kernel(user_ids, user_locations, user_ages, id_table, location_table, age_table):
    info = plsc.get_sparse_core_info()
    nw = info.num_cores * info.num_subcores
    bpw = BATCH // nw
    nch = bpw // CHUNK

    ids = user_ids.astype(jnp.int32).reshape(nw, nch, CHUNK)
    locs = user_locations.astype(jnp.int32).reshape(nw, nch, CHUNK)
    ages = user_ages.astype(jnp.int32).reshape(nw, nch, CHUNK)

    mesh = plsc.VectorSubcoreMesh(core_axis_name="c", subcore_axis_name="s")

    @functools.partial(
        pl.kernel,
        mesh=mesh,
        compiler_params=pltpu.CompilerParams(use_tc_tiling_on_sc=False),
        out_type=jax.ShapeDtypeStruct((BATCH, 3 * EMBED), jnp.float32),
        scratch_types=[
            pltpu.VMEM((nch, CHUNK), jnp.int32),
            pltpu.VMEM((nch, CHUNK), jnp.int32),
            pltpu.VMEM((nch, CHUNK), jnp.int32),
            pltpu.VMEM((bpw, EMBED), jnp.float32),
            pltpu.VMEM((bpw, EMBED), jnp.float32),
            pltpu.VMEM((bpw, EMBED), jnp.float32),
            pltpu.SemaphoreType.DMA,
        ],
    )
    def run(ids_hbm, locs_hbm, ages_hbm, idt_hbm, loct_hbm, aget_hbm, out_hbm,
            idx0, idx1, idx2, r0, r1, r2, sem):
        wid = lax.axis_index("s") * info.num_cores + lax.axis_index("c")
        base = wid * bpw
        pltpu.sync_copy(ids_hbm.at[wid], idx0)
        pltpu.sync_copy(locs_hbm.at[wid], idx1)
        pltpu.sync_copy(ages_hbm.at[wid], idx2)
        copies = []
        for j in range(nch):
            dst = pl.ds(j * CHUNK, CHUNK)
            copies.append(pltpu.async_copy(idt_hbm.at[idx0.at[j]], r0.at[dst], sem))
            copies.append(pltpu.async_copy(loct_hbm.at[idx1.at[j]], r1.at[dst], sem))
            copies.append(pltpu.async_copy(aget_hbm.at[idx2.at[j]], r2.at[dst], sem))
        for c in copies:
            c.wait()
        rows = pl.ds(base, bpw)
        pltpu.sync_copy(r0, out_hbm.at[rows, pl.ds(0, EMBED)])
        pltpu.sync_copy(r1, out_hbm.at[rows, pl.ds(EMBED, EMBED)])
        pltpu.sync_copy(r2, out_hbm.at[rows, pl.ds(2 * EMBED, EMBED)])

    return run(ids, locs, ages, id_table, location_table, age_table)
